# baseline (speedup 1.0000x reference)
import functools

import jax
import jax.numpy as jnp
from jax import lax
from jax.experimental import pallas as pl
from jax.experimental.pallas import tpu as pltpu
from jax.experimental.pallas import tpu_sc as plsc

IN_DIM = 128
OUT_DIM = 128
NUM_RELS = 8
NUM_BASES = 4
N_NODES = 10000
N_EDGES = 320000

NUM_CORES = 2
NUM_SUBCORES = 16
NW = NUM_CORES * NUM_SUBCORES
K = 128
C0 = 62
C1 = 98
E_PAD = NUM_SUBCORES * (C0 + C1) * K
N_ACC = 10112
RPS = N_ACC // NUM_SUBCORES


def _comb_body(wc_ref, wf_ref, out_ref):
    out_ref[...] = jnp.dot(wc_ref[...], wf_ref[...],
                           preferred_element_type=jnp.float32)


def _combine(w_comp, wr_flat):
    return pl.pallas_call(
        _comb_body,
        out_shape=jax.ShapeDtypeStruct((NUM_RELS, IN_DIM * OUT_DIM),
                                       jnp.float32),
    )(w_comp, wr_flat)


def _mm_body(h_ref, w_ref, out_ref):
    out_ref[...] = jnp.dot(h_ref[...], w_ref[...],
                           preferred_element_type=jnp.float32)


def _matmul(h, w2):
    bm = 1000
    return pl.pallas_call(
        _mm_body,
        grid=(N_NODES // bm,),
        in_specs=[
            pl.BlockSpec((bm, IN_DIM), lambda i: (i, 0)),
            pl.BlockSpec((IN_DIM, NUM_RELS * OUT_DIM), lambda i: (0, 0)),
        ],
        out_specs=pl.BlockSpec((bm, NUM_RELS * OUT_DIM), lambda i: (i, 0)),
        out_shape=jax.ShapeDtypeStruct((N_NODES, NUM_RELS * OUT_DIM),
                                       jnp.float32),
    )(h, w2)


def _sc_body(hw_ref, src_ref, et_ref, dst_ref, zero_ref, out_ref,
             srcv, etv, dstv, idxv, rows, acc, sem):
    cid = lax.axis_index("c")
    sid = lax.axis_index("s")

    pltpu.sync_copy(zero_ref.at[pl.ds(sid * RPS, RPS)],
                    acc.at[pl.ds(sid * RPS, RPS)])
    plsc.subcore_barrier()

    nchunks = jnp.where(cid == 0, C0, C1)
    cbase = jnp.where(cid == 0, sid * C0,
                      NUM_SUBCORES * C0 + sid * C1)
    base = cbase * K

    def chunk(j, carry):
        off = base + j * K
        pltpu.sync_copy(src_ref.at[pl.ds(off, K)], srcv)
        pltpu.sync_copy(et_ref.at[pl.ds(off, K)], etv)
        pltpu.sync_copy(dst_ref.at[pl.ds(off, K)], dstv)
        for i in range(K // 16):
            s = pl.ds(i * 16, 16)
            idxv[s] = srcv[s] * NUM_RELS + etv[s]
        pltpu.async_copy(hw_ref.at[idxv], rows, sem).wait()
        pltpu.sync_copy(rows, acc.at[dstv], add=True)
        return carry

    lax.fori_loop(0, nchunks, chunk, 0)
    plsc.subcore_barrier()

    pltpu.sync_copy(acc.at[pl.ds(sid * RPS, RPS)],
                    out_ref.at[pl.ds(cid * N_ACC + sid * RPS, RPS)])


@functools.partial(
    pl.kernel,
    out_type=jax.ShapeDtypeStruct((NUM_CORES * N_ACC, OUT_DIM), jnp.float32),
    mesh=plsc.VectorSubcoreMesh(core_axis_name="c", subcore_axis_name="s"),
    scratch_types=[
        pltpu.VMEM((K,), jnp.int32),
        pltpu.VMEM((K,), jnp.int32),
        pltpu.VMEM((K,), jnp.int32),
        pltpu.VMEM((K,), jnp.int32),
        pltpu.VMEM((K, OUT_DIM), jnp.float32),
        pltpu.VMEM_SHARED((N_ACC, OUT_DIM), jnp.float32),
        pltpu.SemaphoreType.DMA,
    ],
)
def _sc_gather_scatter(hw_ref, src_ref, et_ref, dst_ref, zero_ref, out_ref,
                       srcv, etv, dstv, idxv, rows, acc, sem):
    _sc_body(hw_ref, src_ref, et_ref, dst_ref, zero_ref, out_ref,
             srcv, etv, dstv, idxv, rows, acc, sem)


def kernel(h, edge_index, edge_type, weight, w_comp):
    wr_flat = weight.reshape(IN_DIM, NUM_BASES, OUT_DIM)
    wr_flat = wr_flat.transpose(1, 0, 2).reshape(NUM_BASES, IN_DIM * OUT_DIM)
    wc_perm = _combine(w_comp.astype(jnp.float32), wr_flat)
    w_rel = wc_perm.reshape(NUM_RELS, IN_DIM, OUT_DIM).transpose(1, 0, 2)
    w_rel = w_rel.reshape(NUM_RELS, IN_DIM, OUT_DIM)
    w2 = w_rel.transpose(1, 0, 2).reshape(IN_DIM, NUM_RELS * OUT_DIM)

    hw = _matmul(h, w2).reshape(N_NODES * NUM_RELS, OUT_DIM)

    src = edge_index[0].astype(jnp.int32)
    dst = edge_index[1].astype(jnp.int32)
    et = edge_type.astype(jnp.int32)
    pad = E_PAD - N_EDGES
    src = jnp.concatenate([src, jnp.zeros((pad,), jnp.int32)])
    et = jnp.concatenate([et, jnp.zeros((pad,), jnp.int32)])
    dst = jnp.concatenate([dst, jnp.full((pad,), N_NODES, jnp.int32)])
    zeros = jnp.zeros((N_ACC, OUT_DIM), jnp.float32)

    parts = _sc_gather_scatter(hw, src, et, dst, zeros)
    parts = parts.reshape(NUM_CORES, N_ACC, OUT_DIM)
    return (parts[0] + parts[1])[:N_NODES]


# speedup vs baseline: 1.1550x; 1.1550x over previous
import functools

import jax
import jax.numpy as jnp
from jax import lax
from jax.experimental import pallas as pl
from jax.experimental.pallas import tpu as pltpu
from jax.experimental.pallas import tpu_sc as plsc

IN_DIM = 128
OUT_DIM = 128
NUM_RELS = 8
NUM_BASES = 4
N_NODES = 10000
N_EDGES = 320000

NUM_CORES = 2
NUM_SUBCORES = 16
NW = NUM_CORES * NUM_SUBCORES
K = 128
C0 = 98
C1 = 62
E_PAD = NUM_SUBCORES * (C0 + C1) * K
N_ACC = 10112
RPS = N_ACC // NUM_SUBCORES


def _comb_body(wc_ref, wf_ref, out_ref):
    out_ref[...] = jnp.dot(wc_ref[...], wf_ref[...],
                           preferred_element_type=jnp.float32)


def _combine(w_comp, wr_flat):
    return pl.pallas_call(
        _comb_body,
        out_shape=jax.ShapeDtypeStruct((NUM_RELS, IN_DIM * OUT_DIM),
                                       jnp.float32),
    )(w_comp, wr_flat)


def _mm_body(h_ref, w_ref, out_ref):
    out_ref[...] = jnp.dot(h_ref[...], w_ref[...],
                           preferred_element_type=jnp.float32)


def _matmul(h, w2):
    bm = 1000
    return pl.pallas_call(
        _mm_body,
        grid=(N_NODES // bm,),
        in_specs=[
            pl.BlockSpec((bm, IN_DIM), lambda i: (i, 0)),
            pl.BlockSpec((IN_DIM, NUM_RELS * OUT_DIM), lambda i: (0, 0)),
        ],
        out_specs=pl.BlockSpec((bm, NUM_RELS * OUT_DIM), lambda i: (i, 0)),
        out_shape=jax.ShapeDtypeStruct((N_NODES, NUM_RELS * OUT_DIM),
                                       jnp.float32),
    )(h, w2)


def _sc_body(hw_ref, src_ref, et_ref, dst_ref, zero_ref, out_ref,
             srcv, etv, dstv, idxv, rows, acc, sem):
    cid = lax.axis_index("c")
    sid = lax.axis_index("s")

    pltpu.sync_copy(zero_ref.at[pl.ds(sid * RPS, RPS)],
                    acc.at[pl.ds(sid * RPS, RPS)])
    plsc.subcore_barrier()

    nchunks = jnp.where(cid == 0, C0, C1)
    cbase = jnp.where(cid == 0, sid * C0,
                      NUM_SUBCORES * C0 + sid * C1)
    base = cbase * K

    def chunk(j, carry):
        off = base + j * K
        pltpu.sync_copy(src_ref.at[pl.ds(off, K)], srcv)
        pltpu.sync_copy(et_ref.at[pl.ds(off, K)], etv)
        pltpu.sync_copy(dst_ref.at[pl.ds(off, K)], dstv)
        for i in range(K // 16):
            s = pl.ds(i * 16, 16)
            idxv[s] = srcv[s] * NUM_RELS + etv[s]
        pltpu.async_copy(hw_ref.at[idxv], rows, sem).wait()
        pltpu.sync_copy(rows, acc.at[dstv], add=True)
        return carry

    lax.fori_loop(0, nchunks, chunk, 0)
    plsc.subcore_barrier()

    pltpu.sync_copy(acc.at[pl.ds(sid * RPS, RPS)],
                    out_ref.at[pl.ds(cid * N_ACC + sid * RPS, RPS)])


@functools.partial(
    pl.kernel,
    out_type=jax.ShapeDtypeStruct((NUM_CORES * N_ACC, OUT_DIM), jnp.float32),
    mesh=plsc.VectorSubcoreMesh(core_axis_name="c", subcore_axis_name="s"),
    scratch_types=[
        pltpu.VMEM((K,), jnp.int32),
        pltpu.VMEM((K,), jnp.int32),
        pltpu.VMEM((K,), jnp.int32),
        pltpu.VMEM((K,), jnp.int32),
        pltpu.VMEM((K, OUT_DIM), jnp.float32),
        pltpu.VMEM_SHARED((N_ACC, OUT_DIM), jnp.float32),
        pltpu.SemaphoreType.DMA,
    ],
)
def _sc_gather_scatter(hw_ref, src_ref, et_ref, dst_ref, zero_ref, out_ref,
                       srcv, etv, dstv, idxv, rows, acc, sem):
    _sc_body(hw_ref, src_ref, et_ref, dst_ref, zero_ref, out_ref,
             srcv, etv, dstv, idxv, rows, acc, sem)


def kernel(h, edge_index, edge_type, weight, w_comp):
    wr_flat = weight.reshape(IN_DIM, NUM_BASES, OUT_DIM)
    wr_flat = wr_flat.transpose(1, 0, 2).reshape(NUM_BASES, IN_DIM * OUT_DIM)
    wc_perm = _combine(w_comp.astype(jnp.float32), wr_flat)
    w_rel = wc_perm.reshape(NUM_RELS, IN_DIM, OUT_DIM).transpose(1, 0, 2)
    w_rel = w_rel.reshape(NUM_RELS, IN_DIM, OUT_DIM)
    w2 = w_rel.transpose(1, 0, 2).reshape(IN_DIM, NUM_RELS * OUT_DIM)

    hw = _matmul(h, w2).reshape(N_NODES * NUM_RELS, OUT_DIM)

    src = edge_index[0].astype(jnp.int32)
    dst = edge_index[1].astype(jnp.int32)
    et = edge_type.astype(jnp.int32)
    pad = E_PAD - N_EDGES
    src = jnp.concatenate([src, jnp.zeros((pad,), jnp.int32)])
    et = jnp.concatenate([et, jnp.zeros((pad,), jnp.int32)])
    dst = jnp.concatenate([dst, jnp.full((pad,), N_NODES, jnp.int32)])
    zeros = jnp.zeros((N_ACC, OUT_DIM), jnp.float32)

    parts = _sc_gather_scatter(hw, src, et, dst, zeros)
    parts = parts.reshape(NUM_CORES, N_ACC, OUT_DIM)
    return (parts[0] + parts[1])[:N_NODES]


# speedup vs baseline: 1.1550x; 1.0001x over previous
import functools

import jax
import jax.numpy as jnp
from jax import lax
from jax.experimental import pallas as pl
from jax.experimental.pallas import tpu as pltpu
from jax.experimental.pallas import tpu_sc as plsc

IN_DIM = 128
OUT_DIM = 128
NUM_RELS = 8
NUM_BASES = 4
N_NODES = 10000
N_EDGES = 320000

NUM_CORES = 2
NUM_SUBCORES = 16
NW = NUM_CORES * NUM_SUBCORES
K = 128
C0 = 98
C1 = 62
E_PAD = NUM_SUBCORES * (C0 + C1) * K
N_ACC = 10112
RPS = N_ACC // NUM_SUBCORES


def _comb_body(wc_ref, wf_ref, out_ref):
    out_ref[...] = jnp.dot(wc_ref[...], wf_ref[...],
                           preferred_element_type=jnp.float32)


def _combine(w_comp, wr_flat):
    return pl.pallas_call(
        _comb_body,
        out_shape=jax.ShapeDtypeStruct((NUM_RELS, IN_DIM * OUT_DIM),
                                       jnp.float32),
    )(w_comp, wr_flat)


def _mm_body(h_ref, w_ref, out_ref):
    out_ref[...] = jnp.dot(h_ref[...], w_ref[...],
                           preferred_element_type=jnp.float32)


def _matmul(h, w2):
    bm = 1000
    return pl.pallas_call(
        _mm_body,
        grid=(N_NODES // bm,),
        in_specs=[
            pl.BlockSpec((bm, IN_DIM), lambda i: (i, 0)),
            pl.BlockSpec((IN_DIM, NUM_RELS * OUT_DIM), lambda i: (0, 0)),
        ],
        out_specs=pl.BlockSpec((bm, NUM_RELS * OUT_DIM), lambda i: (i, 0)),
        out_shape=jax.ShapeDtypeStruct((N_NODES, NUM_RELS * OUT_DIM),
                                       jnp.float32),
    )(h, w2)


def _sc_body(hw_ref, src_ref, et_ref, dst_ref, zero_ref, out_ref,
             srcv, etv, dstv, idxv, rows, acc, sem):
    cid = lax.axis_index("c")
    sid = lax.axis_index("s")

    pltpu.sync_copy(zero_ref.at[pl.ds(sid * RPS, RPS)],
                    acc.at[pl.ds(sid * RPS, RPS)])
    plsc.subcore_barrier()

    def run_chunks(base):
        def chunk(j, carry):
            off = base + j * K
            pltpu.sync_copy(src_ref.at[pl.ds(off, K)], srcv)
            pltpu.sync_copy(et_ref.at[pl.ds(off, K)], etv)
            pltpu.sync_copy(dst_ref.at[pl.ds(off, K)], dstv)
            for i in range(K // 16):
                s = pl.ds(i * 16, 16)
                idxv[s] = srcv[s] * NUM_RELS + etv[s]
            pltpu.async_copy(hw_ref.at[idxv], rows, sem).wait()
            pltpu.sync_copy(rows, acc.at[dstv], add=True)
            return carry
        return chunk

    @pl.when(cid == 0)
    def _():
        lax.fori_loop(0, C0, run_chunks(sid * C0 * K), 0)

    @pl.when(cid == 1)
    def _():
        lax.fori_loop(0, C1, run_chunks((NUM_SUBCORES * C0 + sid * C1) * K),
                      0)
    plsc.subcore_barrier()

    pltpu.sync_copy(acc.at[pl.ds(sid * RPS, RPS)],
                    out_ref.at[pl.ds(cid * N_ACC + sid * RPS, RPS)])


@functools.partial(
    pl.kernel,
    out_type=jax.ShapeDtypeStruct((NUM_CORES * N_ACC, OUT_DIM), jnp.float32),
    mesh=plsc.VectorSubcoreMesh(core_axis_name="c", subcore_axis_name="s"),
    scratch_types=[
        pltpu.VMEM((K,), jnp.int32),
        pltpu.VMEM((K,), jnp.int32),
        pltpu.VMEM((K,), jnp.int32),
        pltpu.VMEM((K,), jnp.int32),
        pltpu.VMEM((K, OUT_DIM), jnp.float32),
        pltpu.VMEM_SHARED((N_ACC, OUT_DIM), jnp.float32),
        pltpu.SemaphoreType.DMA,
    ],
)
def _sc_gather_scatter(hw_ref, src_ref, et_ref, dst_ref, zero_ref, out_ref,
                       srcv, etv, dstv, idxv, rows, acc, sem):
    _sc_body(hw_ref, src_ref, et_ref, dst_ref, zero_ref, out_ref,
             srcv, etv, dstv, idxv, rows, acc, sem)


def kernel(h, edge_index, edge_type, weight, w_comp):
    wr_flat = weight.reshape(IN_DIM, NUM_BASES, OUT_DIM)
    wr_flat = wr_flat.transpose(1, 0, 2).reshape(NUM_BASES, IN_DIM * OUT_DIM)
    wc_perm = _combine(w_comp.astype(jnp.float32), wr_flat)
    w_rel = wc_perm.reshape(NUM_RELS, IN_DIM, OUT_DIM).transpose(1, 0, 2)
    w_rel = w_rel.reshape(NUM_RELS, IN_DIM, OUT_DIM)
    w2 = w_rel.transpose(1, 0, 2).reshape(IN_DIM, NUM_RELS * OUT_DIM)

    hw = _matmul(h, w2).reshape(N_NODES * NUM_RELS, OUT_DIM)

    src = edge_index[0].astype(jnp.int32)
    dst = edge_index[1].astype(jnp.int32)
    et = edge_type.astype(jnp.int32)
    pad = E_PAD - N_EDGES
    src = jnp.concatenate([src, jnp.zeros((pad,), jnp.int32)])
    et = jnp.concatenate([et, jnp.zeros((pad,), jnp.int32)])
    dst = jnp.concatenate([dst, jnp.full((pad,), N_NODES, jnp.int32)])
    zeros = jnp.zeros((N_ACC, OUT_DIM), jnp.float32)

    parts = _sc_gather_scatter(hw, src, et, dst, zeros)
    parts = parts.reshape(NUM_CORES, N_ACC, OUT_DIM)
    return (parts[0] + parts[1])[:N_NODES]


# speedup vs baseline: 1.4811x; 1.2823x over previous
import functools

import jax
import jax.numpy as jnp
from jax import lax
from jax.experimental import pallas as pl
from jax.experimental.pallas import tpu as pltpu
from jax.experimental.pallas import tpu_sc as plsc

IN_DIM = 128
OUT_DIM = 128
NUM_RELS = 8
NUM_BASES = 4
N_NODES = 10000
N_EDGES = 320000

NUM_CORES = 2
NUM_SUBCORES = 16
NW = NUM_CORES * NUM_SUBCORES
K = 128
CHUNKS = -(-N_EDGES // (NW * K))
PER_W = CHUNKS * K
E_PAD = NW * PER_W
N_ACC = 10112
RPS = N_ACC // NUM_SUBCORES


def _comb_body(wc_ref, wf_ref, out_ref):
    out_ref[...] = jnp.dot(wc_ref[...], wf_ref[...],
                           preferred_element_type=jnp.float32)


def _combine(w_comp, wr_flat):
    return pl.pallas_call(
        _comb_body,
        out_shape=jax.ShapeDtypeStruct((NUM_RELS, IN_DIM * OUT_DIM),
                                       jnp.float32),
    )(w_comp, wr_flat)


def _mm_body(h_ref, w_ref, out_ref):
    out_ref[...] = jnp.dot(h_ref[...], w_ref[...],
                           preferred_element_type=jnp.float32)


def _matmul(h, w2):
    bm = 1000
    return pl.pallas_call(
        _mm_body,
        grid=(N_NODES // bm,),
        in_specs=[
            pl.BlockSpec((bm, IN_DIM), lambda i: (i, 0)),
            pl.BlockSpec((IN_DIM, NUM_RELS * OUT_DIM), lambda i: (0, 0)),
        ],
        out_specs=pl.BlockSpec((bm, NUM_RELS * OUT_DIM), lambda i: (i, 0)),
        out_shape=jax.ShapeDtypeStruct((N_NODES, NUM_RELS * OUT_DIM),
                                       jnp.float32),
    )(h, w2)


def _sc_body(hw_ref, src_ref, et_ref, dst_ref, zero_ref, out_ref,
             srcv, etv, dstv, idxv, rows, acc, sem):
    cid = lax.axis_index("c")
    sid = lax.axis_index("s")
    wid = cid * NUM_SUBCORES + sid

    pltpu.sync_copy(zero_ref.at[pl.ds(sid * RPS, RPS)],
                    acc.at[pl.ds(sid * RPS, RPS)])
    plsc.subcore_barrier()

    base = wid * PER_W

    def chunk(j, carry):
        off = base + j * K
        pltpu.sync_copy(src_ref.at[pl.ds(off, K)], srcv)
        pltpu.sync_copy(et_ref.at[pl.ds(off, K)], etv)
        pltpu.sync_copy(dst_ref.at[pl.ds(off, K)], dstv)
        for i in range(K // 16):
            s = pl.ds(i * 16, 16)
            idxv[s] = srcv[s] * NUM_RELS + etv[s]
        pltpu.async_copy(hw_ref.at[idxv], rows, sem).wait()
        pltpu.sync_copy(rows, acc.at[dstv], add=True)
        return carry

    lax.fori_loop(0, CHUNKS, chunk, 0)
    plsc.subcore_barrier()

    pltpu.sync_copy(acc.at[pl.ds(sid * RPS, RPS)],
                    out_ref.at[pl.ds(cid * N_ACC + sid * RPS, RPS)])


@functools.partial(
    pl.kernel,
    out_type=jax.ShapeDtypeStruct((NUM_CORES * N_ACC, OUT_DIM), jnp.float32),
    mesh=plsc.VectorSubcoreMesh(core_axis_name="c", subcore_axis_name="s"),
    scratch_types=[
        pltpu.VMEM((K,), jnp.int32),
        pltpu.VMEM((K,), jnp.int32),
        pltpu.VMEM((K,), jnp.int32),
        pltpu.VMEM((K,), jnp.int32),
        pltpu.VMEM((K, OUT_DIM), jnp.float32),
        pltpu.VMEM_SHARED((N_ACC, OUT_DIM), jnp.float32),
        pltpu.SemaphoreType.DMA,
    ],
)
def _sc_gather_scatter(hw_ref, src_ref, et_ref, dst_ref, zero_ref, out_ref,
                       srcv, etv, dstv, idxv, rows, acc, sem):
    _sc_body(hw_ref, src_ref, et_ref, dst_ref, zero_ref, out_ref,
             srcv, etv, dstv, idxv, rows, acc, sem)


def kernel(h, edge_index, edge_type, weight, w_comp):
    wr_flat = weight.reshape(IN_DIM, NUM_BASES, OUT_DIM)
    wr_flat = wr_flat.transpose(1, 0, 2).reshape(NUM_BASES, IN_DIM * OUT_DIM)
    wc_perm = _combine(w_comp.astype(jnp.float32), wr_flat)
    w_rel = wc_perm.reshape(NUM_RELS, IN_DIM, OUT_DIM).transpose(1, 0, 2)
    w_rel = w_rel.reshape(NUM_RELS, IN_DIM, OUT_DIM)
    w2 = w_rel.transpose(1, 0, 2).reshape(IN_DIM, NUM_RELS * OUT_DIM)

    hw = _matmul(h, w2).reshape(N_NODES * NUM_RELS, OUT_DIM)

    src = edge_index[0].astype(jnp.int32)
    dst = edge_index[1].astype(jnp.int32)
    et = edge_type.astype(jnp.int32)
    pad = E_PAD - N_EDGES
    src = jnp.concatenate([src, jnp.zeros((pad,), jnp.int32)])
    et = jnp.concatenate([et, jnp.zeros((pad,), jnp.int32)])
    dst = jnp.concatenate([dst, jnp.full((pad,), N_NODES, jnp.int32)])
    zeros = jnp.zeros((N_ACC, OUT_DIM), jnp.float32)

    parts = _sc_gather_scatter(hw, src, et, dst, zeros)
    parts = parts.reshape(NUM_CORES, N_ACC, OUT_DIM)
    return (parts[0] + parts[1])[:N_NODES]
